# baseline (speedup 1.0000x reference)
import functools

import jax
import jax.numpy as jnp
from jax import lax
from jax.experimental import pallas as pl
from jax.experimental.pallas import tpu as pltpu
from jax.experimental.pallas import tpu_sc as plsc

N = 50000
D = 128
K = 16
HID = 16
C = 40
NP = 50176
RT = NP // 16
CW = 32
NCH = D // CW
NEP = 25088
EB = 8
NB = NEP // EB
PB1 = NB // 16
PB2 = NB // 32
INV_KM1 = 1.0 / (K - 1)
BR = 512
BF = 400


def _nsqrt(x):
    xi = plsc.bitcast(x, jnp.int32)
    y = plsc.bitcast(jnp.int32(0x5F3759DF) - (xi >> 1), jnp.float32)
    xh = 0.5 * x
    y = y * (1.5 - xh * y * y)
    y = y * (1.5 - xh * y * y)
    return x * y


GB = 28
NG = PB1 // GB


def _sc1_body(hp, hc, en, out, acc, idx_all, idxo_a, idxo_b, rows_a, rows_b,
              contrib, sem_a, sem_b):
    c = lax.axis_index("c")
    s = lax.axis_index("s")

    z16 = jnp.zeros((16,), jnp.float32)

    def mk_off(i, dst, base):
        for v in range(8):
            sl = pl.ds(v * 16, 16)
            dst[sl] = idx_all[i, sl] + base

    def compute_scatter(i, rows):
        def edge(e, _):
            r0 = e * K

            def ksum(kk, tt):
                a0, a1 = tt
                for u in range(4):
                    r = r0 + kk * 4 + u
                    a0 = a0 + rows[r, pl.ds(0, 16)]
                    a1 = a1 + rows[r, pl.ds(16, 16)]
                return (a0, a1)

            t0, t1 = lax.fori_loop(0, 4, ksum, (z16, z16))

            def kcon(kk, _):
                for u in range(4):
                    r = r0 + kk * 4 + u
                    contrib[r, pl.ds(0, 16)] = _nsqrt(
                        (t0 - rows[r, pl.ds(0, 16)]) * INV_KM1)
                    contrib[r, pl.ds(16, 16)] = _nsqrt(
                        (t1 - rows[r, pl.ds(16, 16)]) * INV_KM1)
                return 0

            lax.fori_loop(0, 4, kcon, 0)
            return 0

        lax.fori_loop(0, EB, edge, 0)
        pltpu.sync_copy(contrib, acc.at[idx_all.at[i]], add=True)

    def chunk_pass(j, _):
        chunk = c * 2 + j
        base = chunk * NP
        pltpu.sync_copy(hc.at[pl.ds(base + s * RT, RT)], acc.at[pl.ds(s * RT, RT)])
        plsc.subcore_barrier()

        def group(g, _):
            pltpu.sync_copy(en.at[pl.ds(s * PB1 + g * GB, GB)], idx_all)
            mk_off(0, idxo_a, base)
            pltpu.async_copy(hp.at[idxo_a], rows_a, sem_a)

            def pair(it, _):
                i = it * 2
                mk_off(i + 1, idxo_b, base)
                pltpu.async_copy(hp.at[idxo_b], rows_b, sem_b)
                pltpu.make_async_copy(hp.at[idxo_a], rows_a, sem_a).wait()
                compute_scatter(i, rows_a)

                @pl.when(it + 1 < GB // 2)
                def _():
                    mk_off(i + 2, idxo_a, base)
                    pltpu.async_copy(hp.at[idxo_a], rows_a, sem_a)

                pltpu.make_async_copy(hp.at[idxo_b], rows_b, sem_b).wait()
                compute_scatter(i + 1, rows_b)
                return 0

            lax.fori_loop(0, GB // 2, pair, 0)
            return 0

        lax.fori_loop(0, NG, group, 0)
        plsc.subcore_barrier()
        pltpu.sync_copy(acc.at[pl.ds(s * RT, RT)], out.at[pl.ds(base + s * RT, RT)])
        plsc.subcore_barrier()
        return 0

    lax.fori_loop(0, 2, chunk_pass, 0)


def _sc2_body(h1p, h1ch, en, out, acc, idx_all, rows_a, rows_b, contrib,
              sem_a, sem_b):
    c = lax.axis_index("c")
    s = lax.axis_index("s")
    pltpu.sync_copy(en.at[pl.ds(c * (NB // 2) + s * PB2, PB2)], idx_all)
    pltpu.async_copy(h1p.at[idx_all.at[0]], rows_a, sem_a)
    pltpu.sync_copy(h1ch.at[pl.ds(s * RT, RT)], acc.at[pl.ds(s * RT, RT)])
    plsc.subcore_barrier()

    z16 = jnp.zeros((16,), jnp.float32)

    def compute_scatter(i, rows):
        def edge(e, _):
            r0 = e * K

            def ksum(kk, a):
                for u in range(4):
                    a = a + rows[r0 + kk * 4 + u, :]
                return a

            t = lax.fori_loop(0, 4, ksum, z16)

            def kcon(kk, _):
                for u in range(4):
                    r = r0 + kk * 4 + u
                    contrib[r, :] = _nsqrt((t - rows[r, :]) * INV_KM1)
                return 0

            lax.fori_loop(0, 4, kcon, 0)
            return 0

        lax.fori_loop(0, EB, edge, 0)
        pltpu.sync_copy(contrib, acc.at[idx_all.at[i]], add=True)

    def pair(it, _):
        i = it * 2
        pltpu.async_copy(h1p.at[idx_all.at[i + 1]], rows_b, sem_b)
        pltpu.make_async_copy(h1p.at[idx_all.at[i]], rows_a, sem_a).wait()
        compute_scatter(i, rows_a)

        @pl.when(it + 1 < PB2 // 2)
        def _():
            pltpu.async_copy(h1p.at[idx_all.at[i + 2]], rows_a, sem_a)

        pltpu.make_async_copy(h1p.at[idx_all.at[i + 1]], rows_b, sem_b).wait()
        compute_scatter(i + 1, rows_b)
        return 0

    lax.fori_loop(0, PB2 // 2, pair, 0)
    plsc.subcore_barrier()
    pltpu.sync_copy(acc.at[pl.ds(s * RT, RT)], out.at[pl.ds(c * NP + s * RT, RT)])


def _prep_body(h_ref, hc_ref, hp_ref):
    x = jnp.clip(h_ref[...], 1e-7, 10.0)
    for ch in range(NCH):
        xc = x[:, ch * CW:(ch + 1) * CW]
        hc_ref[ch] = xc
        hp_ref[ch] = xc * xc


def _mid_body(l1_ref, w1_ref, b1_ref, h1p_ref, h1ch_ref):
    x = l1_ref[...]
    rs = jnp.sum(x, axis=(0, 2))
    h = jnp.dot(x[0], w1_ref[0], preferred_element_type=jnp.float32)
    for ch in range(1, NCH):
        h = h + jnp.dot(x[ch], w1_ref[ch], preferred_element_type=jnp.float32)
    rinv = 1.0 / rs
    rinv = jnp.where(jnp.isinf(rinv), 0.0, rinv)
    h1 = jnp.maximum(h * rinv[:, None] + b1_ref[...], 0.0)
    h1c = jnp.clip(h1, 1e-7, 10.0)
    h1p_ref[...] = h1c * h1c
    h1ch_ref[...] = 0.5 * h1c


def _fin_body(p_ref, w2_ref, b2_ref, out_ref):
    p = p_ref[...]
    ah = p[0] + p[1]
    rs = jnp.sum(ah, axis=1)
    rinv = 1.0 / rs
    rinv = jnp.where(jnp.isinf(rinv), 0.0, rinv)
    out_ref[...] = (jnp.dot(ah, w2_ref[...], preferred_element_type=jnp.float32)
                    * rinv[:, None] + b2_ref[...])


@functools.lru_cache(maxsize=None)
def _sc_kernels():
    mesh = plsc.VectorSubcoreMesh(
        core_axis_name="c", subcore_axis_name="s", num_cores=2, num_subcores=16)
    params = pltpu.CompilerParams(
        needs_layout_passes=False, use_tc_tiling_on_sc=False)
    sc1 = pl.kernel(
        _sc1_body,
        out_type=jax.ShapeDtypeStruct((NCH * NP, CW), jnp.float32),
        mesh=mesh,
        compiler_params=params,
        scratch_types=[
            pltpu.VMEM_SHARED((NP, CW), jnp.float32),
            pltpu.VMEM((GB, 128), jnp.int32),
            pltpu.VMEM((128,), jnp.int32),
            pltpu.VMEM((128,), jnp.int32),
            pltpu.VMEM((128, CW), jnp.float32),
            pltpu.VMEM((128, CW), jnp.float32),
            pltpu.VMEM((128, CW), jnp.float32),
            pltpu.SemaphoreType.DMA,
            pltpu.SemaphoreType.DMA,
        ])
    sc2 = pl.kernel(
        _sc2_body,
        out_type=jax.ShapeDtypeStruct((2 * NP, HID), jnp.float32),
        mesh=mesh,
        compiler_params=params,
        scratch_types=[
            pltpu.VMEM_SHARED((NP, HID), jnp.float32),
            pltpu.VMEM((PB2, 128), jnp.int32),
            pltpu.VMEM((128, HID), jnp.float32),
            pltpu.VMEM((128, HID), jnp.float32),
            pltpu.VMEM((128, HID), jnp.float32),
            pltpu.SemaphoreType.DMA,
            pltpu.SemaphoreType.DMA,
        ])
    return sc1, sc2


def kernel(H, edge_nodes, W1, b1, W2, b2):
    f32 = jnp.float32
    sc1, sc2 = _sc_kernels()
    ne = edge_nodes.shape[0]
    Hpad = jnp.concatenate([H.astype(f32), jnp.zeros((NP - N, D), f32)], axis=0)
    en = jnp.concatenate(
        [edge_nodes.astype(jnp.int32),
         jnp.full((NEP - ne, K), N, jnp.int32)], axis=0).reshape(NB, EB * K)

    hc4, hp4 = pl.pallas_call(
        _prep_body,
        grid=(NP // BR,),
        in_specs=[pl.BlockSpec((BR, D), lambda i: (i, 0))],
        out_specs=[pl.BlockSpec((NCH, BR, CW), lambda i: (0, i, 0))] * 2,
        out_shape=[jax.ShapeDtypeStruct((NCH, NP, CW), f32)] * 2,
    )(Hpad)

    l1 = sc1(hp4.reshape(NCH * NP, CW), hc4.reshape(NCH * NP, CW), en)

    h1p, h1ch = pl.pallas_call(
        _mid_body,
        grid=(NP // BR,),
        in_specs=[pl.BlockSpec((NCH, BR, CW), lambda i: (0, i, 0)),
                  pl.BlockSpec((NCH, CW, HID), lambda i: (0, 0, 0)),
                  pl.BlockSpec((1, HID), lambda i: (0, 0))],
        out_specs=[pl.BlockSpec((BR, HID), lambda i: (i, 0))] * 2,
        out_shape=[jax.ShapeDtypeStruct((NP, HID), f32)] * 2,
    )(l1.reshape(NCH, NP, CW), W1.astype(f32).reshape(NCH, CW, HID),
      b1.astype(f32).reshape(1, HID))

    l2 = sc2(h1p, h1ch, en)

    out = pl.pallas_call(
        _fin_body,
        grid=(N // BF,),
        in_specs=[pl.BlockSpec((2, BF, HID), lambda i: (0, i, 0)),
                  pl.BlockSpec((HID, C), lambda i: (0, 0)),
                  pl.BlockSpec((1, C), lambda i: (0, 0))],
        out_specs=pl.BlockSpec((BF, C), lambda i: (i, 0)),
        out_shape=jax.ShapeDtypeStruct((N, C), f32),
    )(l2.reshape(2, NP, HID), W2.astype(f32), b2.astype(f32).reshape(1, C))
    return out


# speedup vs baseline: 1.8205x; 1.8205x over previous
import functools

import jax
import jax.numpy as jnp
from jax import lax
from jax.experimental import pallas as pl
from jax.experimental.pallas import tpu as pltpu
from jax.experimental.pallas import tpu_sc as plsc

N = 50000
D = 128
K = 16
HID = 16
C = 40
NP = 50176
RT = NP // 16
CW = 32
NCH = D // CW
NEP = 25088
EB = 8
NB = NEP // EB
PB1 = NB // 16
PB2 = NB // 32
INV_KM1 = 1.0 / (K - 1)
BR = 512
BF = 400


def _nsqrt(x):
    xi = plsc.bitcast(x, jnp.int32)
    y = plsc.bitcast(jnp.int32(0x5F3759DF) - (xi >> 1), jnp.float32)
    xh = 0.5 * x
    y = y * (1.5 - xh * y * y)
    y = y * (1.5 - xh * y * y)
    return x * y


def _sc1_body(hp, hc, en, out, acc, idxr_a, idxr_b, idxo_a, idxo_b,
              rows_a, rows_b, contrib, sem_a, sem_b):
    c = lax.axis_index("c")
    s = lax.axis_index("s")

    z16 = jnp.zeros((16,), jnp.float32)

    def fetch(i, idxr, idxo, rows, sem, base):
        b = i * 16 + s
        pltpu.sync_copy(en.at[pl.ds(b * 128, 128)], idxr)
        for v in range(8):
            sl = pl.ds(v * 16, 16)
            idxo[sl] = idxr[sl] + base
        pltpu.async_copy(hp.at[idxo], rows, sem)

    def compute_scatter(idxr, rows):
        for e in range(EB):
            r0 = e * K

            def ksum(kk, tt):
                return (tt[0] + rows[r0 + kk, pl.ds(0, 16)],
                        tt[1] + rows[r0 + kk, pl.ds(16, 16)])

            t0, t1 = lax.fori_loop(0, K, ksum, (z16, z16))

            def kcon(kk, _):
                contrib[r0 + kk, pl.ds(0, 16)] = _nsqrt(
                    (t0 - rows[r0 + kk, pl.ds(0, 16)]) * INV_KM1)
                contrib[r0 + kk, pl.ds(16, 16)] = _nsqrt(
                    (t1 - rows[r0 + kk, pl.ds(16, 16)]) * INV_KM1)
                return 0

            lax.fori_loop(0, K, kcon, 0)
        pltpu.sync_copy(contrib, acc.at[idxr], add=True)

    def chunk_pass(j, _):
        chunk = c * 2 + j
        base = chunk * NP
        pltpu.sync_copy(hc.at[pl.ds(base + s * RT, RT)], acc.at[pl.ds(s * RT, RT)])
        fetch(0, idxr_a, idxo_a, rows_a, sem_a, base)
        plsc.subcore_barrier()

        def pair(it, _):
            i = it * 2
            fetch(i + 1, idxr_b, idxo_b, rows_b, sem_b, base)
            pltpu.make_async_copy(hp.at[idxo_a], rows_a, sem_a).wait()
            compute_scatter(idxr_a, rows_a)

            @pl.when(it + 1 < PB1 // 2)
            def _():
                fetch(i + 2, idxr_a, idxo_a, rows_a, sem_a, base)

            pltpu.make_async_copy(hp.at[idxo_b], rows_b, sem_b).wait()
            compute_scatter(idxr_b, rows_b)
            return 0

        lax.fori_loop(0, PB1 // 2, pair, 0)
        plsc.subcore_barrier()
        pltpu.sync_copy(acc.at[pl.ds(s * RT, RT)], out.at[pl.ds(base + s * RT, RT)])
        plsc.subcore_barrier()
        return 0

    lax.fori_loop(0, 2, chunk_pass, 0)


def _sc2_body(h1p, h1ch, en, out, acc, idxr_a, idxr_b, rows_a, rows_b, contrib,
              sem_a, sem_b):
    c = lax.axis_index("c")
    s = lax.axis_index("s")

    z16 = jnp.zeros((16,), jnp.float32)

    def fetch(i, idxr, rows, sem):
        b = c * (NB // 2) + i * 16 + s
        pltpu.sync_copy(en.at[pl.ds(b * 128, 128)], idxr)
        pltpu.async_copy(h1p.at[idxr], rows, sem)

    def compute_scatter(idxr, rows):
        for e in range(EB):
            r0 = e * K

            def ksum(kk, a):
                return a + rows[r0 + kk, :]

            t = lax.fori_loop(0, K, ksum, z16)

            def kcon(kk, _):
                contrib[r0 + kk, :] = _nsqrt((t - rows[r0 + kk, :]) * INV_KM1)
                return 0

            lax.fori_loop(0, K, kcon, 0)
        pltpu.sync_copy(contrib, acc.at[idxr], add=True)

    fetch(0, idxr_a, rows_a, sem_a)
    pltpu.sync_copy(h1ch.at[pl.ds(s * RT, RT)], acc.at[pl.ds(s * RT, RT)])
    plsc.subcore_barrier()

    def pair(it, _):
        i = it * 2
        fetch(i + 1, idxr_b, rows_b, sem_b)
        pltpu.make_async_copy(h1p.at[idxr_a], rows_a, sem_a).wait()
        compute_scatter(idxr_a, rows_a)

        @pl.when(it + 1 < PB2 // 2)
        def _():
            fetch(i + 2, idxr_a, rows_a, sem_a)

        pltpu.make_async_copy(h1p.at[idxr_b], rows_b, sem_b).wait()
        compute_scatter(idxr_b, rows_b)
        return 0

    lax.fori_loop(0, PB2 // 2, pair, 0)
    plsc.subcore_barrier()
    pltpu.sync_copy(acc.at[pl.ds(s * RT, RT)], out.at[pl.ds(c * NP + s * RT, RT)])


def _prep_body(h_ref, hc_ref, hp_ref):
    x = jnp.clip(h_ref[...], 1e-7, 10.0)
    for ch in range(NCH):
        xc = x[:, ch * CW:(ch + 1) * CW]
        hc_ref[ch] = xc
        hp_ref[ch] = xc * xc


def _mid_body(l1_ref, w1_ref, b1_ref, h1p_ref, h1ch_ref):
    x = l1_ref[...]
    rs = jnp.sum(x, axis=(0, 2))
    h = jnp.dot(x[0], w1_ref[0], preferred_element_type=jnp.float32)
    for ch in range(1, NCH):
        h = h + jnp.dot(x[ch], w1_ref[ch], preferred_element_type=jnp.float32)
    rinv = 1.0 / rs
    rinv = jnp.where(jnp.isinf(rinv), 0.0, rinv)
    h1 = jnp.maximum(h * rinv[:, None] + b1_ref[...], 0.0)
    h1c = jnp.clip(h1, 1e-7, 10.0)
    h1p_ref[...] = h1c * h1c
    h1ch_ref[...] = 0.5 * h1c


def _fin_body(p_ref, w2_ref, b2_ref, out_ref):
    p = p_ref[...]
    ah = p[0] + p[1]
    rs = jnp.sum(ah, axis=1)
    rinv = 1.0 / rs
    rinv = jnp.where(jnp.isinf(rinv), 0.0, rinv)
    out_ref[...] = (jnp.dot(ah, w2_ref[...], preferred_element_type=jnp.float32)
                    * rinv[:, None] + b2_ref[...])


@functools.lru_cache(maxsize=None)
def _sc_kernels():
    mesh = plsc.VectorSubcoreMesh(
        core_axis_name="c", subcore_axis_name="s", num_cores=2, num_subcores=16)
    params = pltpu.CompilerParams(
        needs_layout_passes=False, use_tc_tiling_on_sc=False)
    sc1 = pl.kernel(
        _sc1_body,
        out_type=jax.ShapeDtypeStruct((NCH * NP, CW), jnp.float32),
        mesh=mesh,
        compiler_params=params,
        scratch_types=[
            pltpu.VMEM_SHARED((NP, CW), jnp.float32),
            pltpu.VMEM((128,), jnp.int32),
            pltpu.VMEM((128,), jnp.int32),
            pltpu.VMEM((128,), jnp.int32),
            pltpu.VMEM((128,), jnp.int32),
            pltpu.VMEM((128, CW), jnp.float32),
            pltpu.VMEM((128, CW), jnp.float32),
            pltpu.VMEM((128, CW), jnp.float32),
            pltpu.SemaphoreType.DMA,
            pltpu.SemaphoreType.DMA,
        ])
    sc2 = pl.kernel(
        _sc2_body,
        out_type=jax.ShapeDtypeStruct((2 * NP, HID), jnp.float32),
        mesh=mesh,
        compiler_params=params,
        scratch_types=[
            pltpu.VMEM_SHARED((NP, HID), jnp.float32),
            pltpu.VMEM((128,), jnp.int32),
            pltpu.VMEM((128,), jnp.int32),
            pltpu.VMEM((128, HID), jnp.float32),
            pltpu.VMEM((128, HID), jnp.float32),
            pltpu.VMEM((128, HID), jnp.float32),
            pltpu.SemaphoreType.DMA,
            pltpu.SemaphoreType.DMA,
        ])
    return sc1, sc2


def kernel(H, edge_nodes, W1, b1, W2, b2):
    f32 = jnp.float32
    sc1, sc2 = _sc_kernels()
    ne = edge_nodes.shape[0]
    Hpad = jnp.concatenate([H.astype(f32), jnp.zeros((NP - N, D), f32)], axis=0)
    en = jnp.concatenate(
        [edge_nodes.astype(jnp.int32),
         jnp.full((NEP - ne, K), N, jnp.int32)], axis=0).reshape(-1)

    hc4, hp4 = pl.pallas_call(
        _prep_body,
        grid=(NP // BR,),
        in_specs=[pl.BlockSpec((BR, D), lambda i: (i, 0))],
        out_specs=[pl.BlockSpec((NCH, BR, CW), lambda i: (0, i, 0))] * 2,
        out_shape=[jax.ShapeDtypeStruct((NCH, NP, CW), f32)] * 2,
    )(Hpad)

    l1 = sc1(hp4.reshape(NCH * NP, CW), hc4.reshape(NCH * NP, CW), en)

    h1p, h1ch = pl.pallas_call(
        _mid_body,
        grid=(NP // BR,),
        in_specs=[pl.BlockSpec((NCH, BR, CW), lambda i: (0, i, 0)),
                  pl.BlockSpec((NCH, CW, HID), lambda i: (0, 0, 0)),
                  pl.BlockSpec((1, HID), lambda i: (0, 0))],
        out_specs=[pl.BlockSpec((BR, HID), lambda i: (i, 0))] * 2,
        out_shape=[jax.ShapeDtypeStruct((NP, HID), f32)] * 2,
    )(l1.reshape(NCH, NP, CW), W1.astype(f32).reshape(NCH, CW, HID),
      b1.astype(f32).reshape(1, HID))

    l2 = sc2(h1p, h1ch, en)

    out = pl.pallas_call(
        _fin_body,
        grid=(N // BF,),
        in_specs=[pl.BlockSpec((2, BF, HID), lambda i: (0, i, 0)),
                  pl.BlockSpec((HID, C), lambda i: (0, 0)),
                  pl.BlockSpec((1, C), lambda i: (0, 0))],
        out_specs=pl.BlockSpec((BF, C), lambda i: (i, 0)),
        out_shape=jax.ShapeDtypeStruct((N, C), f32),
    )(l2.reshape(2, NP, HID), W2.astype(f32), b2.astype(f32).reshape(1, C))
    return out
